# baseline (speedup 1.0000x reference)
import functools

import jax
import jax.numpy as jnp
from jax import lax
from jax.experimental import pallas as pl
from jax.experimental.pallas import tpu as pltpu
from jax.experimental.pallas import tpu_sc as plsc

D_MODEL = 64
_SCALE = 8.0

_NC = 2
_NS = 16
_NW = _NC * _NS
_L = 16

_C = 128
_P = 2
_K = 2


@functools.lru_cache(maxsize=None)
def _build(n_slab: int, vocab: int):
    nunit = n_slab // _P
    nblk = nunit // _K
    assert nblk * _K == nunit
    n1t = n_slab // 8

    mesh = plsc.VectorSubcoreMesh(
        core_axis_name="c", subcore_axis_name="s",
        num_cores=_NC, num_subcores=_NS,
    )

    @functools.partial(
        pl.kernel,
        mesh=mesh,
        out_type=jax.ShapeDtypeStruct((n_slab, 8, _NW, 8, _C), jnp.float32),
        scratch_types=[
            pltpu.VMEM((n1t, 8, _C), jnp.int32),
            [pltpu.VMEM((_P, _C, D_MODEL), jnp.float32) for _ in range(_K)],
            [pltpu.VMEM((_P, 8, 8, _C + 1), jnp.float32) for _ in range(_K)],
            [pltpu.SemaphoreType.DMA for _ in range(_K)],
            [pltpu.SemaphoreType.DMA for _ in range(_K)],
        ],
        compiler_params=pltpu.CompilerParams(
            use_tc_tiling_on_sc=False, needs_layout_passes=False
        ),
    )
    def kern(x_hbm, ta_hbm, tb_hbm, out_hbm, idx_v, gbufs, tbufs, gsems, osems):
        cid = lax.axis_index("c")
        sid = lax.axis_index("s")
        wid = sid * _NC + cid

        pltpu.sync_copy(x_hbm.at[:, wid], idx_v)

        def start_gather(g, b):
            for h in range(_P):
                a = g * _P + h

                @pl.when(cid == 0)
                def _():
                    pltpu.async_copy(
                        ta_hbm.at[idx_v.at[a // 8, a % 8]],
                        gbufs[b].at[h], gsems[b],
                    )

                @pl.when(cid == 1)
                def _():
                    pltpu.async_copy(
                        tb_hbm.at[idx_v.at[a // 8, a % 8]],
                        gbufs[b].at[h], gsems[b],
                    )

        def wait_gather(b):
            for h in range(_P):
                pltpu.make_async_copy(
                    ta_hbm.at[idx_v.at[0, 0]], gbufs[b].at[h], gsems[b]
                ).wait()

        def start_out(g, b):
            pltpu.async_copy(
                tbufs[b].at[:, :, :, pl.ds(0, _C)],
                out_hbm.at[pl.ds(g * _P, _P), :, wid],
                osems[b],
            )

        def wait_out(b):
            pltpu.make_async_copy(
                tbufs[b].at[:, :, :, pl.ds(0, _C)],
                out_hbm.at[pl.ds(0, _P), :, 0],
                osems[b],
            ).wait()

        lane = lax.iota(jnp.int32, _L)
        bbv = [(lane + k * _L) // 8 for k in range(D_MODEL // _L)]
        bsv = [(lane + k * _L) % 8 for k in range(D_MODEL // _L)]

        def transpose_scale(b):
            gbuf = gbufs[b]
            tbuf = tbufs[b]

            @pl.loop(0, _C, unroll=2)
            def _(r):
                rv = jnp.full((_L,), r, jnp.int32)
                for h in range(_P):
                    hv = jnp.full((_L,), h, jnp.int32)
                    vals = [
                        gbuf[h, r, pl.ds(k * _L, _L)] * _SCALE
                        for k in range(D_MODEL // _L)
                    ]
                    for k in range(D_MODEL // _L):
                        plsc.store_scatter(
                            tbuf, [hv, bbv[k], bsv[k], rv], vals[k]
                        )

        def step(g, b, *, first=False, prefetch=True):
            if not first:
                wait_out((b + _K - 1) % _K)
            if prefetch:
                start_gather(g + _K - 1, (b + _K - 1) % _K)
            wait_gather(b)
            transpose_scale(b)
            start_out(g, b)

        for b in range(_K - 1):
            start_gather(b, b)

        step(0, 0, first=True)
        for b in range(1, _K):
            step(b, b)

        @pl.loop(1, nblk - 1)
        def _(i):
            g0 = i * _K
            for b in range(_K):
                step(g0 + b, b)

        g0 = (nblk - 1) * _K
        step(g0, 0)
        for b in range(1, _K):
            step(g0 + b, b, prefetch=False)

        wait_out(_K - 1)

    return kern


def kernel(x, table):
    s0, s1 = x.shape
    vocab = table.shape[0]
    x5 = (
        x.T.reshape(s1 // 8, 8, s0 // _C, _C)
        .transpose(0, 2, 1, 3)
        .astype(jnp.int32)
    )
    table_b = lax.optimization_barrier(table)
    out5 = _build(s1, vocab)(x5, table, table_b)
    return out5.transpose(2, 4, 0, 1, 3).reshape(s0, s1, D_MODEL)


# speedup vs baseline: 1.1564x; 1.1564x over previous
import functools

import jax
import jax.numpy as jnp
from jax import lax
from jax.experimental import pallas as pl
from jax.experimental.pallas import tpu as pltpu
from jax.experimental.pallas import tpu_sc as plsc

D_MODEL = 64
_SCALE = 8.0

_NC = 2
_NS = 16
_NW = _NC * _NS
_L = 16

_C = 128
_P = 2
_K = 2


@functools.lru_cache(maxsize=None)
def _build(n_slab: int, vocab: int):
    nunit = n_slab // _P
    nblk = nunit // _K
    assert nblk * _K == nunit
    n1t = n_slab // 8

    mesh = plsc.VectorSubcoreMesh(
        core_axis_name="c", subcore_axis_name="s",
        num_cores=_NC, num_subcores=_NS,
    )

    @functools.partial(
        pl.kernel,
        mesh=mesh,
        out_type=jax.ShapeDtypeStruct((n_slab, 8, _NW, 8, _C), jnp.float32),
        scratch_types=[
            pltpu.VMEM((n1t, 8, _C), jnp.int32),
            [pltpu.VMEM((_P, _C, D_MODEL), jnp.float32) for _ in range(_K)],
            [pltpu.VMEM((_P, 8, 8, _C + 1), jnp.float32) for _ in range(_K)],
            [pltpu.SemaphoreType.DMA for _ in range(_K)],
            [pltpu.SemaphoreType.DMA for _ in range(_K)],
        ],
        compiler_params=pltpu.CompilerParams(
            use_tc_tiling_on_sc=False, needs_layout_passes=False
        ),
    )
    def kern(x_hbm, table_hbm, out_hbm, idx_v, gbufs, tbufs, gsems, osems):
        cid = lax.axis_index("c")
        sid = lax.axis_index("s")
        wid = sid * _NC + cid

        pltpu.sync_copy(x_hbm.at[:, wid], idx_v)

        def start_gather(g, b):
            for h in range(_P):
                a = g * _P + h
                pltpu.async_copy(
                    table_hbm.at[idx_v.at[a // 8, a % 8]],
                    gbufs[b].at[h], gsems[b],
                )

        def wait_gather(b):
            for h in range(_P):
                pltpu.make_async_copy(
                    table_hbm.at[idx_v.at[0, 0]], gbufs[b].at[h], gsems[b]
                ).wait()

        def start_out(g, b):
            pltpu.async_copy(
                tbufs[b].at[:, :, :, pl.ds(0, _C)],
                out_hbm.at[pl.ds(g * _P, _P), :, wid],
                osems[b],
            )

        def wait_out(b):
            pltpu.make_async_copy(
                tbufs[b].at[:, :, :, pl.ds(0, _C)],
                out_hbm.at[pl.ds(0, _P), :, 0],
                osems[b],
            ).wait()

        lane = lax.iota(jnp.int32, _L)
        bbv = [(lane + k * _L) // 8 for k in range(D_MODEL // _L)]
        bsv = [(lane + k * _L) % 8 for k in range(D_MODEL // _L)]

        def transpose_scale(b):
            gbuf = gbufs[b]
            tbuf = tbufs[b]

            @pl.loop(0, _C, unroll=2)
            def _(r):
                rv = jnp.full((_L,), r, jnp.int32)
                for h in range(_P):
                    hv = jnp.full((_L,), h, jnp.int32)
                    vals = [
                        gbuf[h, r, pl.ds(k * _L, _L)]
                        for k in range(D_MODEL // _L)
                    ]
                    for k in range(D_MODEL // _L):
                        plsc.store_scatter(
                            tbuf, [hv, bbv[k], bsv[k], rv], vals[k]
                        )

        def step(g, b, *, first=False, prefetch=True):
            if not first:
                wait_out((b + _K - 1) % _K)
            if prefetch:
                start_gather(g + _K - 1, (b + _K - 1) % _K)
            wait_gather(b)
            transpose_scale(b)
            start_out(g, b)

        for b in range(_K - 1):
            start_gather(b, b)

        step(0, 0, first=True)
        for b in range(1, _K):
            step(b, b)

        @pl.loop(1, nblk - 1)
        def _(i):
            g0 = i * _K
            for b in range(_K):
                step(g0 + b, b)

        g0 = (nblk - 1) * _K
        step(g0, 0)
        for b in range(1, _K):
            step(g0 + b, b, prefetch=False)

        wait_out(_K - 1)

    return kern


def kernel(x, table):
    s0, s1 = x.shape
    vocab = table.shape[0]
    x5 = (
        x.T.reshape(s1 // 8, 8, s0 // _C, _C)
        .transpose(0, 2, 1, 3)
        .astype(jnp.int32)
    )
    table8 = table * jnp.float32(_SCALE)
    out5 = _build(s1, vocab)(x5, table8)
    return out5.transpose(2, 4, 0, 1, 3).reshape(s0, s1, D_MODEL)


# speedup vs baseline: 1.5092x; 1.3051x over previous
import functools

import jax
import jax.numpy as jnp
from jax import lax
from jax.experimental import pallas as pl
from jax.experimental.pallas import tpu as pltpu
from jax.experimental.pallas import tpu_sc as plsc

D_MODEL = 64
_SCALE = 8.0

_NC = 2
_NS = 16
_NW = _NC * _NS
_L = 16

_C = 128
_P = 2
_K = 2


@functools.lru_cache(maxsize=None)
def _build(n_slab: int, vocab: int):
    nunit = n_slab // _P
    nblk = nunit // _K
    assert nblk * _K == nunit
    n1t = n_slab // 8

    mesh = plsc.VectorSubcoreMesh(
        core_axis_name="c", subcore_axis_name="s",
        num_cores=_NC, num_subcores=_NS,
    )

    @functools.partial(
        pl.kernel,
        mesh=mesh,
        out_type=jax.ShapeDtypeStruct((n_slab, 8, _NW, 8, _C), jnp.float32),
        scratch_types=[
            pltpu.VMEM((n1t, 8, _C), jnp.int32),
            [pltpu.VMEM((_P, _C, D_MODEL), jnp.float32) for _ in range(_K)],
            [pltpu.VMEM((_P, 8, 8, _C + 1), jnp.float32) for _ in range(_K)],
            [pltpu.SemaphoreType.DMA for _ in range(_K)],
            [pltpu.SemaphoreType.DMA for _ in range(_K)],
        ],
        compiler_params=pltpu.CompilerParams(
            use_tc_tiling_on_sc=False, needs_layout_passes=False
        ),
    )
    def kern(x_hbm, table_hbm, out_hbm, idx_v, gbufs, tbufs, gsems, osems):
        cid = lax.axis_index("c")
        sid = lax.axis_index("s")
        wid = sid * _NC + cid

        pltpu.sync_copy(x_hbm.at[:, wid], idx_v)

        def start_gather(g, b):
            for h in range(_P):
                a = g * _P + h
                pltpu.async_copy(
                    table_hbm.at[idx_v.at[a // 8, a % 8]],
                    gbufs[b].at[h], gsems[b],
                )

        def wait_gather(b):
            for h in range(_P):
                pltpu.make_async_copy(
                    table_hbm.at[idx_v.at[0, 0]], gbufs[b].at[h], gsems[b]
                ).wait()

        def start_out(g, b):
            pltpu.async_copy(
                tbufs[b].at[:, :, :, pl.ds(0, _C)],
                out_hbm.at[pl.ds(g * _P, _P), :, wid],
                osems[b],
            )

        def wait_out(b):
            pltpu.make_async_copy(
                tbufs[b].at[:, :, :, pl.ds(0, _C)],
                out_hbm.at[pl.ds(0, _P), :, 0],
                osems[b],
            ).wait()

        lane = lax.iota(jnp.int32, _L)
        bbv = [(lane + k * _L) // 8 for k in range(D_MODEL // _L)]
        bsv = [(lane + k * _L) % 8 for k in range(D_MODEL // _L)]

        def transpose_scale(b):
            gbuf = gbufs[b]
            tbuf = tbufs[b]

            @pl.loop(0, _C, unroll=2)
            def _(r):
                rv = jnp.full((_L,), r, jnp.int32)
                for h in range(_P):
                    hv = jnp.full((_L,), h, jnp.int32)
                    vals = [
                        gbuf[h, r, pl.ds(k * _L, _L)] * _SCALE
                        for k in range(D_MODEL // _L)
                    ]
                    for k in range(D_MODEL // _L):
                        plsc.store_scatter(
                            tbuf, [hv, bbv[k], bsv[k], rv], vals[k]
                        )

        def step(g, b, *, first=False, prefetch=True):
            if not first:
                wait_out((b + _K - 1) % _K)
            if prefetch:
                start_gather(g + _K - 1, (b + _K - 1) % _K)
            wait_gather(b)
            transpose_scale(b)
            start_out(g, b)

        for b in range(_K - 1):
            start_gather(b, b)

        step(0, 0, first=True)
        for b in range(1, _K):
            step(b, b)

        @pl.loop(1, nblk - 1)
        def _(i):
            g0 = i * _K
            for b in range(_K):
                step(g0 + b, b)

        g0 = (nblk - 1) * _K
        step(g0, 0)
        for b in range(1, _K):
            step(g0 + b, b, prefetch=False)

        wait_out(_K - 1)

    return kern


def kernel(x, table):
    s0, s1 = x.shape
    vocab = table.shape[0]
    x5 = (
        x.T.reshape(s1 // 8, 8, s0 // _C, _C)
        .transpose(0, 2, 1, 3)
        .astype(jnp.int32)
    )
    out5 = _build(s1, vocab)(x5, table)
    return out5.transpose(2, 4, 0, 1, 3).reshape(s0, s1, D_MODEL)
